# baseline (speedup 1.0000x reference)
import functools

import jax
import jax.numpy as jnp
import numpy as np
from jax import lax
from jax.experimental import pallas as pl
from jax.experimental.pallas import tpu as pltpu
from jax.experimental.pallas import tpu_sc as plsc

BS = 2; L = 4; C = 256; H = 128; W = 128; K = 1024
NH = 8; NP = 4; NLAYERS = 3; DFF = 512; DH = C // NH
CLASS_COUNTS = [1, 2, 2, 1, 2, 2]
HW = H * W
NCORNER = 4
RPT = NH * L * NP * NCORNER
LPG = NH * L * NP
NW = 32
PER_W = BS * K // NW


@functools.lru_cache(maxsize=1)
def _qpos_table():
    eps = 1e-6
    nf = C // 2
    ye = (np.arange(H, dtype=np.float32) + 1.0) / (H + eps) * 2 * np.pi
    xe = (np.arange(W, dtype=np.float32) + 1.0) / (W + eps) * 2 * np.pi
    dt = np.arange(nf, dtype=np.float32)
    dt = (10000.0 ** (2 * (dt // 2) / nf)).astype(np.float32)
    px = xe[:, None] / dt
    py = ye[:, None] / dt
    px = np.stack([np.sin(px[:, 0::2]), np.cos(px[:, 1::2])], 2).reshape(W, -1)
    py = np.stack([np.sin(py[:, 0::2]), np.cos(py[:, 1::2])], 2).reshape(H, -1)
    tab = np.concatenate(
        [np.broadcast_to(py[:, None, :], (H, W, nf)),
         np.broadcast_to(px[None, :, :], (H, W, nf))], axis=2)
    return tab.reshape(HW, C).astype(np.float32)


_PBV = 2048


def _v_body(x_ref, wv_ref, vb_ref, o0_ref, o1_ref, o2_ref):
    xb = x_ref[0, 0]
    for n, o_ref in enumerate((o0_ref, o1_ref, o2_ref)):
        acc = lax.dot_general(xb, wv_ref[n], (((0,), (0,)), ((), ())),
                              preferred_element_type=jnp.float32)
        acc = acc + vb_ref[n, 0, 0][None, :]
        accb = acc.astype(jnp.bfloat16)
        for h in range(NH):
            o_ref[0, 0, h] = accb[:, h * DH:(h + 1) * DH]


def _v_project(x4b, Wvb, vbias):
    ospec = pl.BlockSpec((1, 1, NH, _PBV, DH), lambda b, l, j: (b, l, 0, j, 0))
    oshape = jax.ShapeDtypeStruct((BS, L, NH, HW, DH), jnp.bfloat16)
    return pl.pallas_call(
        _v_body,
        grid=(BS, L, HW // _PBV),
        in_specs=[
            pl.BlockSpec((1, 1, C, _PBV), lambda b, l, j: (b, l, 0, j)),
            pl.BlockSpec((NLAYERS, C, C), lambda b, l, j: (0, 0, 0)),
            pl.BlockSpec((NLAYERS, 1, 1, C), lambda b, l, j: (0, l, 0, 0)),
        ],
        out_specs=[ospec, ospec, ospec],
        out_shape=[oshape, oshape, oshape],
    )(x4b, Wvb, vbias)


def _prep_block(q, ind, wox, box, woy, boy, waw, baw, b):
    offx = jnp.dot(q, wox, preferred_element_type=jnp.float32) + box[0][None, :]
    offy = jnp.dot(q, woy, preferred_element_type=jnp.float32) + boy[0][None, :]
    a = jnp.dot(q, waw, preferred_element_type=jnp.float32) + baw[0][None, :]
    a = jax.nn.softmax(a.reshape(K, NH, L * NP), -1).reshape(K, NH * L * NP)
    rx = (ind % W).astype(jnp.float32).reshape(K, 1)
    ry = (ind // W).astype(jnp.float32).reshape(K, 1)
    refx = (rx + 0.5) / W
    refy = (ry + 0.5) / H
    px = (refx + offx / W) * W - 0.5
    py = (refy + offy / H) * H - 0.5
    x0 = jnp.floor(px); y0 = jnp.floor(py)
    lane = lax.broadcasted_iota(jnp.int32, (K, NH * L * NP), 1)
    l_of = (lane // NP) % L
    h_of = lane // (L * NP)
    base = ((b * L + l_of) * NH + h_of) * HW
    idxs = []; wgts = []
    for dy in (0, 1):
        for dx in (0, 1):
            xc = x0 + dx; yc = y0 + dy
            wx = 1.0 - jnp.abs(px - xc)
            wy = 1.0 - jnp.abs(py - yc)
            valid = (xc >= 0) & (xc < W) & (yc >= 0) & (yc < H)
            wc = jnp.where(valid, wx * wy, 0.0) * a
            xi = jnp.clip(xc, 0, W - 1).astype(jnp.int32)
            yi = jnp.clip(yc, 0, H - 1).astype(jnp.int32)
            idxs.append(base + yi * W + xi)
            wgts.append(wc)
    return jnp.concatenate(idxs, 1), jnp.concatenate(wgts, 1)


def _t1_body(qsel_ref, qpos_ref, ind_ref, w1_ref, b1_ref, w2_ref, b2_ref,
             wox_ref, box_ref, woy_ref, boy_ref, waw_ref, baw_ref,
             query_ref, idx_ref, wgt_ref):
    b = pl.program_id(0)
    q70 = qsel_ref[0]
    h1 = jax.nn.gelu(jnp.dot(q70, w1_ref[...],
                             preferred_element_type=jnp.float32)
                     + b1_ref[0][None, :])
    query = jnp.dot(h1, w2_ref[...],
                    preferred_element_type=jnp.float32) + b2_ref[0][None, :]
    query_ref[0] = query
    q = query + qpos_ref[0]
    idx, wgt = _prep_block(q, ind_ref[0, 0, :], wox_ref[...], box_ref,
                           woy_ref[...], boy_ref, waw_ref[...], baw_ref, b)
    idx_ref[0] = idx
    wgt_ref[0] = wgt


def _t1(qsel80, qpos, ind3, w1p, b1, w2, b2, wox, box, woy, boy, waw, baw):
    wspec = lambda shape: pl.BlockSpec(shape, lambda b: tuple(0 for _ in shape))
    return pl.pallas_call(
        _t1_body,
        grid=(BS,),
        in_specs=[
            pl.BlockSpec((1, K, 80), lambda b: (b, 0, 0)),
            pl.BlockSpec((1, K, C), lambda b: (b, 0, 0)),
            pl.BlockSpec((1, 1, K), lambda b: (b, 0, 0)),
            wspec((80, C)), wspec((1, C)), wspec((C, C)), wspec((1, C)),
            wspec((C, LPG)), wspec((1, LPG)), wspec((C, LPG)), wspec((1, LPG)),
            wspec((C, LPG)), wspec((1, LPG)),
        ],
        out_specs=[
            pl.BlockSpec((1, K, C), lambda b: (b, 0, 0)),
            pl.BlockSpec((1, K, RPT), lambda b: (b, 0, 0)),
            pl.BlockSpec((1, K, RPT), lambda b: (b, 0, 0)),
        ],
        out_shape=[
            jax.ShapeDtypeStruct((BS, K, C), jnp.float32),
            jax.ShapeDtypeStruct((BS, K, RPT), jnp.int32),
            jax.ShapeDtypeStruct((BS, K, RPT), jnp.float32),
        ],
    )(qsel80, qpos, ind3, w1p, b1, w2, b2, wox, box, woy, boy, waw, baw)


def _ln_rows(x, g, b, eps=1e-5):
    m = x.mean(-1, keepdims=True)
    v = ((x - m) ** 2).mean(-1, keepdims=True)
    return (x - m) / jnp.sqrt(v + eps) * g[0][None, :] + b[0][None, :]


def _t2_body(query_ref, qpos_ref, ind_ref, acc_ref, wo_ref, bo_ref,
             g1_ref, be1_ref, f1_ref, fb1_ref, f2_ref, fb2_ref, g2_ref,
             be2_ref, wox_ref, box_ref, woy_ref, boy_ref, waw_ref, baw_ref,
             query_out_ref, idx_ref, wgt_ref):
    b = pl.program_id(0)
    out = jnp.dot(acc_ref[0], wo_ref[...],
                  preferred_element_type=jnp.float32) + bo_ref[0][None, :]
    query = _ln_rows(query_ref[0] + out, g1_ref, be1_ref)
    hdd = jnp.dot(jax.nn.relu(
        jnp.dot(query, f1_ref[...], preferred_element_type=jnp.float32)
        + fb1_ref[0][None, :]), f2_ref[...],
        preferred_element_type=jnp.float32) + fb2_ref[0][None, :]
    query = _ln_rows(query + hdd, g2_ref, be2_ref)
    query_out_ref[0] = query
    q = query + qpos_ref[0]
    idx, wgt = _prep_block(q, ind_ref[0, 0, :], wox_ref[...], box_ref,
                           woy_ref[...], boy_ref, waw_ref[...], baw_ref, b)
    idx_ref[0] = idx
    wgt_ref[0] = wgt


def _t2(query, qpos, ind3, acc, wo, bo, g1, be1, f1, fb1, f2, fb2, g2, be2,
        wox, box, woy, boy, waw, baw):
    wspec = lambda shape: pl.BlockSpec(shape, lambda b: tuple(0 for _ in shape))
    return pl.pallas_call(
        _t2_body,
        grid=(BS,),
        in_specs=[
            pl.BlockSpec((1, K, C), lambda b: (b, 0, 0)),
            pl.BlockSpec((1, K, C), lambda b: (b, 0, 0)),
            pl.BlockSpec((1, 1, K), lambda b: (b, 0, 0)),
            pl.BlockSpec((1, K, C), lambda b: (b, 0, 0)),
            wspec((C, C)), wspec((1, C)),
            wspec((1, C)), wspec((1, C)),
            wspec((C, DFF)), wspec((1, DFF)), wspec((DFF, C)), wspec((1, C)),
            wspec((1, C)), wspec((1, C)),
            wspec((C, LPG)), wspec((1, LPG)), wspec((C, LPG)), wspec((1, LPG)),
            wspec((C, LPG)), wspec((1, LPG)),
        ],
        out_specs=[
            pl.BlockSpec((1, K, C), lambda b: (b, 0, 0)),
            pl.BlockSpec((1, K, RPT), lambda b: (b, 0, 0)),
            pl.BlockSpec((1, K, RPT), lambda b: (b, 0, 0)),
        ],
        out_shape=[
            jax.ShapeDtypeStruct((BS, K, C), jnp.float32),
            jax.ShapeDtypeStruct((BS, K, RPT), jnp.int32),
            jax.ShapeDtypeStruct((BS, K, RPT), jnp.float32),
        ],
    )(query, qpos, ind3, acc, wo, bo, g1, be1, f1, fb1, f2, fb2, g2, be2,
      wox, box, woy, boy, waw, baw)


def _sc_sample(vtab, idx, wgt):
    mesh = plsc.VectorSubcoreMesh(core_axis_name="c", subcore_axis_name="s")

    NBUF = 4

    @functools.partial(
        pl.kernel, mesh=mesh,
        out_type=jax.ShapeDtypeStruct((BS * K, NH, DH), jnp.float32),
        compiler_params=pltpu.CompilerParams(use_tc_tiling_on_sc=False),
        scratch_types=[
            pltpu.VMEM((NBUF, NCORNER, 128), jnp.int32),
            pltpu.VMEM((NBUF, RPT), jnp.float32),
            pltpu.VMEM((NBUF, RPT, DH // 2), jnp.int32),
            pltpu.VMEM((NH, DH), jnp.float32),
        ] + [pltpu.SemaphoreType.DMA] * NBUF,
    )
    def k(vtab_hbm, idx_hbm, wgt_hbm, out_hbm, idx_v, wgt_v, rows_v,
          stage_v, *sems):
        wid = lax.axis_index("s") * 2 + lax.axis_index("c")
        o0 = wid * PER_W

        def fire(o, s):
            pltpu.sync_copy(idx_hbm.at[o], idx_v.at[s])
            pltpu.sync_copy(wgt_hbm.at[o], wgt_v.at[s])
            for j in range(NCORNER):
                pltpu.async_copy(vtab_hbm.at[idx_v.at[s, j]],
                                 rows_v.at[s, pl.ds(j * 128, 128)], sems[s])

        def drain(s):
            for j in range(NCORNER):
                pltpu.make_async_copy(vtab_hbm.at[idx_v.at[s, j]],
                                      rows_v.at[s, pl.ds(j * 128, 128)],
                                      sems[s]).wait()

        def compute(o, s):
            for h in range(NH):
                acc0 = jnp.zeros((16,), jnp.float32)
                acc1 = jnp.zeros((16,), jnp.float32)
                for c in range(NCORNER):
                    wv = wgt_v[s, c * 128 + h * 16:c * 128 + h * 16 + 16]
                    for t in range(L * NP):
                        r = c * 128 + h * 16 + t
                        w = wv[t]
                        ri = rows_v[s, r, :]
                        ra = lax.bitcast_convert_type(
                            lax.shift_left(ri, 16), jnp.float32)
                        rb = lax.bitcast_convert_type(
                            lax.bitwise_and(ri, jnp.int32(-65536)), jnp.float32)
                        acc0 = acc0 + w * ra
                        acc1 = acc1 + w * rb
                stage_v[h, 0:16] = acc0
                stage_v[h, 16:32] = acc1
            pltpu.sync_copy(stage_v, out_hbm.at[o])

        for s in range(NBUF):
            fire(o0 + s, s)

        def body(j, _):
            o = o0 + NBUF * j
            for s in range(NBUF):
                drain(s)
                compute(o + s, s)
                pl.when(j < PER_W // NBUF - 1)(
                    functools.partial(fire, o + s + NBUF, s))
            return 0

        lax.fori_loop(0, PER_W // NBUF, body, 0)

    return k(vtab, idx, wgt)


_PB = 2048


def _scatter_body(ind_ref, q_ref, out_ref):
    j = pl.program_id(1)
    ind = ind_ref[0, 0, :].reshape(K, 1)
    cols = lax.broadcasted_iota(jnp.int32, (K, _PB), 1) + j * _PB
    oh = (cols == ind).astype(jnp.float32)
    out_ref[0] = lax.dot_general(q_ref[0], oh, (((0,), (0,)), ((), ())),
                                 preferred_element_type=jnp.float32)


def _scatter_fill(query, ind3):
    return pl.pallas_call(
        _scatter_body,
        grid=(BS, HW // _PB),
        in_specs=[
            pl.BlockSpec((1, 1, K), lambda b, j: (b, 0, 0)),
            pl.BlockSpec((1, K, C), lambda b, j: (b, 0, 0)),
        ],
        out_specs=pl.BlockSpec((1, C, _PB), lambda b, j: (b, 0, j)),
        out_shape=jax.ShapeDtypeStruct((BS, C, HW), jnp.float32),
    )(ind3, query)


def kernel(x, preds, mlp_w1, mlp_b1, mlp_w2, mlp_b2, time_emb, Wv, bv, Woff,
           boff, Waw, baw, Wo, bo, ln1_g, ln1_b, ffn_w1, ffn_b1, ffn_w2,
           ffn_b2, ln2_g, ln2_b):
    x4 = x.reshape(BS, L, C, HW)
    woff_r = Woff.reshape(NLAYERS, C, NH, L, NP, 2)
    wox = woff_r[..., 0].reshape(NLAYERS, C, LPG)
    woy = woff_r[..., 1].reshape(NLAYERS, C, LPG)
    boff_r = boff.reshape(NLAYERS, NH, L, NP, 2)
    box = boff_r[..., 0].reshape(NLAYERS, 1, LPG)
    boy = boff_r[..., 1].reshape(NLAYERS, 1, LPG)
    vbias = (jnp.einsum('lc,ncd->nld', time_emb, Wv)
             + bv[:, None, :]).reshape(NLAYERS, L, 1, C)
    x4b = x4.astype(jnp.bfloat16)
    Wvb = Wv.astype(jnp.bfloat16)
    j16 = np.arange(16)
    blockperm = np.concatenate([2 * j16, 2 * j16 + 1])
    pfull = (np.repeat(np.arange(NH) * DH, DH)
             + np.tile(blockperm, NH)).astype(np.int32)
    Wo_p = Wo[:, pfull, :]
    w1p = jnp.concatenate([mlp_w1, jnp.zeros((10, C), jnp.float32)], 0)
    b1 = mlp_b1.reshape(1, C)
    b2 = mlp_b2.reshape(1, C)
    baw_r = baw.reshape(NLAYERS, 1, LPG)
    bo_r = bo.reshape(NLAYERS, 1, C)
    ln1g_r = ln1_g.reshape(NLAYERS, 1, C); ln1b_r = ln1_b.reshape(NLAYERS, 1, C)
    ln2g_r = ln2_g.reshape(NLAYERS, 1, C); ln2b_r = ln2_b.reshape(NLAYERS, 1, C)
    fb1_r = ffn_b1.reshape(NLAYERS, 1, DFF); fb2_r = ffn_b2.reshape(NLAYERS, 1, C)

    hm = []; start = 0
    for cc in CLASS_COUNTS:
        hm.append(preds[:, start + 10:start + 10 + cc])
        start += 10 + cc
    heat = jnp.clip(jax.nn.sigmoid(jnp.concatenate(hm, 1)), 1e-4, 1.0 - 1e-4)
    hmask = jnp.max(heat, axis=1).reshape(BS, -1)
    _, mask_ind = lax.top_k(hmask, K)
    ind3 = mask_ind.reshape(BS, 1, K)
    q_all = preds.reshape(BS, 70, -1).transpose(0, 2, 1)
    qsel = jnp.take_along_axis(
        q_all, jnp.broadcast_to(mask_ind[:, :, None], (BS, K, 70)), axis=1)
    qsel80 = jnp.concatenate([qsel, jnp.zeros((BS, K, 10), jnp.float32)], 2)
    qpos = jnp.asarray(_qpos_table())[mask_ind]

    query, idx, wgt = _t1(qsel80, qpos, ind3, w1p, b1, mlp_w2, b2,
                          wox[0], box[0], woy[0], boy[0], Waw[0], baw_r[0])
    vts = _v_project(x4b, Wvb, vbias)
    for lyr in range(NLAYERS):
        vt = lax.bitcast_convert_type(
            vts[lyr].reshape(BS * L * NH * HW, DH // 2, 2),
            jnp.int32)
        acc = _sc_sample(vt, idx.reshape(BS * K, NCORNER, 128),
                         wgt.reshape(BS * K, RPT))
        nxt = min(lyr + 1, NLAYERS - 1)
        query, idx, wgt = _t2(
            query, qpos, ind3, acc.reshape(BS, K, C), Wo_p[lyr], bo_r[lyr],
            ln1g_r[lyr], ln1b_r[lyr], ffn_w1[lyr], fb1_r[lyr], ffn_w2[lyr],
            fb2_r[lyr], ln2g_r[lyr], ln2b_r[lyr], wox[nxt], box[nxt],
            woy[nxt], boy[nxt], Waw[nxt], baw_r[nxt])
    fill = _scatter_fill(query, ind3)
    return fill.reshape(BS, C, H, W)


# speedup vs baseline: 3.0640x; 3.0640x over previous
import functools

import jax
import jax.numpy as jnp
import numpy as np
from jax import lax
from jax.experimental import pallas as pl
from jax.experimental.pallas import tpu as pltpu
from jax.experimental.pallas import tpu_sc as plsc

BS = 2; L = 4; C = 256; H = 128; W = 128; K = 1024
NH = 8; NP = 4; NLAYERS = 3; DFF = 512; DH = C // NH
CLASS_COUNTS = [1, 2, 2, 1, 2, 2]
HW = H * W
NCORNER = 4
RPT = NH * L * NP * NCORNER
LPG = NH * L * NP
NW = 32
PER_W = BS * K // NW


@functools.lru_cache(maxsize=1)
def _qpos_table():
    eps = 1e-6
    nf = C // 2
    ye = (np.arange(H, dtype=np.float32) + 1.0) / (H + eps) * 2 * np.pi
    xe = (np.arange(W, dtype=np.float32) + 1.0) / (W + eps) * 2 * np.pi
    dt = np.arange(nf, dtype=np.float32)
    dt = (10000.0 ** (2 * (dt // 2) / nf)).astype(np.float32)
    px = xe[:, None] / dt
    py = ye[:, None] / dt
    px = np.stack([np.sin(px[:, 0::2]), np.cos(px[:, 1::2])], 2).reshape(W, -1)
    py = np.stack([np.sin(py[:, 0::2]), np.cos(py[:, 1::2])], 2).reshape(H, -1)
    tab = np.concatenate(
        [np.broadcast_to(py[:, None, :], (H, W, nf)),
         np.broadcast_to(px[None, :, :], (H, W, nf))], axis=2)
    return tab.reshape(HW, C).astype(np.float32)


_PBV = 2048


def _rtne_bf16_bits(x):
    b = lax.bitcast_convert_type(x, jnp.int32)
    r = b + 0x7FFF + lax.bitwise_and(lax.shift_right_logical(b, 16), 1)
    return lax.shift_right_logical(r, 16)


def _v_body(x_ref, wv_ref, vb_ref, o0_ref, o1_ref, o2_ref):
    xb = x_ref[0, 0]
    for n, o_ref in enumerate((o0_ref, o1_ref, o2_ref)):
        acc = lax.dot_general(xb, wv_ref[n], (((0,), (0,)), ((), ())),
                              preferred_element_type=jnp.float32)
        acc = acc + vb_ref[n, 0, 0][None, :]
        for h in range(NH):
            lo = _rtne_bf16_bits(acc[:, h * DH:h * DH + DH // 2])
            hi = _rtne_bf16_bits(acc[:, h * DH + DH // 2:(h + 1) * DH])
            o_ref[0, 0, h] = lax.bitwise_or(lo, lax.shift_left(hi, 16))


def _v_project(x4b, Wvb, vbias):
    ospec = pl.BlockSpec((1, 1, NH, _PBV, DH // 2),
                         lambda b, l, j: (b, l, 0, j, 0))
    oshape = jax.ShapeDtypeStruct((BS, L, NH, HW, DH // 2), jnp.int32)
    return pl.pallas_call(
        _v_body,
        grid=(BS, L, HW // _PBV),
        in_specs=[
            pl.BlockSpec((1, 1, C, _PBV), lambda b, l, j: (b, l, 0, j)),
            pl.BlockSpec((NLAYERS, C, C), lambda b, l, j: (0, 0, 0)),
            pl.BlockSpec((NLAYERS, 1, 1, C), lambda b, l, j: (0, l, 0, 0)),
        ],
        out_specs=[ospec, ospec, ospec],
        out_shape=[oshape, oshape, oshape],
    )(x4b, Wvb, vbias)


def _prep_block(q, ind, wox, box, woy, boy, waw, baw, b):
    offx = jnp.dot(q, wox, preferred_element_type=jnp.float32) + box[0][None, :]
    offy = jnp.dot(q, woy, preferred_element_type=jnp.float32) + boy[0][None, :]
    a = jnp.dot(q, waw, preferred_element_type=jnp.float32) + baw[0][None, :]
    a = jax.nn.softmax(a.reshape(K, NH, L * NP), -1).reshape(K, NH * L * NP)
    rx = (ind % W).astype(jnp.float32).reshape(K, 1)
    ry = (ind // W).astype(jnp.float32).reshape(K, 1)
    refx = (rx + 0.5) / W
    refy = (ry + 0.5) / H
    px = (refx + offx / W) * W - 0.5
    py = (refy + offy / H) * H - 0.5
    x0 = jnp.floor(px); y0 = jnp.floor(py)
    lane = lax.broadcasted_iota(jnp.int32, (K, NH * L * NP), 1)
    l_of = (lane // NP) % L
    h_of = lane // (L * NP)
    base = ((b * L + l_of) * NH + h_of) * HW
    idxs = []; wgts = []
    for dy in (0, 1):
        for dx in (0, 1):
            xc = x0 + dx; yc = y0 + dy
            wx = 1.0 - jnp.abs(px - xc)
            wy = 1.0 - jnp.abs(py - yc)
            valid = (xc >= 0) & (xc < W) & (yc >= 0) & (yc < H)
            wc = jnp.where(valid, wx * wy, 0.0) * a
            xi = jnp.clip(xc, 0, W - 1).astype(jnp.int32)
            yi = jnp.clip(yc, 0, H - 1).astype(jnp.int32)
            idxs.append(base + yi * W + xi)
            wgts.append(wc)
    return jnp.concatenate(idxs, 1), jnp.concatenate(wgts, 1)


def _t1_body(qsel_ref, qpos_ref, ind_ref, w1_ref, b1_ref, w2_ref, b2_ref,
             wox_ref, box_ref, woy_ref, boy_ref, waw_ref, baw_ref,
             query_ref, idx_ref, wgt_ref):
    b = pl.program_id(0)
    q70 = qsel_ref[0]
    h1 = jax.nn.gelu(jnp.dot(q70, w1_ref[...],
                             preferred_element_type=jnp.float32)
                     + b1_ref[0][None, :])
    query = jnp.dot(h1, w2_ref[...],
                    preferred_element_type=jnp.float32) + b2_ref[0][None, :]
    query_ref[0] = query
    q = query + qpos_ref[0]
    idx, wgt = _prep_block(q, ind_ref[0, 0, :], wox_ref[...], box_ref,
                           woy_ref[...], boy_ref, waw_ref[...], baw_ref, b)
    idx_ref[0] = idx
    wgt_ref[0] = wgt


def _t1(qsel80, qpos, ind3, w1p, b1, w2, b2, wox, box, woy, boy, waw, baw):
    wspec = lambda shape: pl.BlockSpec(shape, lambda b: tuple(0 for _ in shape))
    return pl.pallas_call(
        _t1_body,
        grid=(BS,),
        in_specs=[
            pl.BlockSpec((1, K, 80), lambda b: (b, 0, 0)),
            pl.BlockSpec((1, K, C), lambda b: (b, 0, 0)),
            pl.BlockSpec((1, 1, K), lambda b: (b, 0, 0)),
            wspec((80, C)), wspec((1, C)), wspec((C, C)), wspec((1, C)),
            wspec((C, LPG)), wspec((1, LPG)), wspec((C, LPG)), wspec((1, LPG)),
            wspec((C, LPG)), wspec((1, LPG)),
        ],
        out_specs=[
            pl.BlockSpec((1, K, C), lambda b: (b, 0, 0)),
            pl.BlockSpec((1, K, RPT), lambda b: (b, 0, 0)),
            pl.BlockSpec((1, K, RPT), lambda b: (b, 0, 0)),
        ],
        out_shape=[
            jax.ShapeDtypeStruct((BS, K, C), jnp.float32),
            jax.ShapeDtypeStruct((BS, K, RPT), jnp.int32),
            jax.ShapeDtypeStruct((BS, K, RPT), jnp.float32),
        ],
    )(qsel80, qpos, ind3, w1p, b1, w2, b2, wox, box, woy, boy, waw, baw)


def _ln_rows(x, g, b, eps=1e-5):
    m = x.mean(-1, keepdims=True)
    v = ((x - m) ** 2).mean(-1, keepdims=True)
    return (x - m) / jnp.sqrt(v + eps) * g[0][None, :] + b[0][None, :]


def _t2_body(query_ref, qpos_ref, ind_ref, acc_ref, wo_ref, bo_ref,
             g1_ref, be1_ref, f1_ref, fb1_ref, f2_ref, fb2_ref, g2_ref,
             be2_ref, wox_ref, box_ref, woy_ref, boy_ref, waw_ref, baw_ref,
             query_out_ref, idx_ref, wgt_ref):
    b = pl.program_id(0)
    out = jnp.dot(acc_ref[0], wo_ref[...],
                  preferred_element_type=jnp.float32) + bo_ref[0][None, :]
    query = _ln_rows(query_ref[0] + out, g1_ref, be1_ref)
    hdd = jnp.dot(jax.nn.relu(
        jnp.dot(query, f1_ref[...], preferred_element_type=jnp.float32)
        + fb1_ref[0][None, :]), f2_ref[...],
        preferred_element_type=jnp.float32) + fb2_ref[0][None, :]
    query = _ln_rows(query + hdd, g2_ref, be2_ref)
    query_out_ref[0] = query
    q = query + qpos_ref[0]
    idx, wgt = _prep_block(q, ind_ref[0, 0, :], wox_ref[...], box_ref,
                           woy_ref[...], boy_ref, waw_ref[...], baw_ref, b)
    idx_ref[0] = idx
    wgt_ref[0] = wgt


def _t2(query, qpos, ind3, acc, wo, bo, g1, be1, f1, fb1, f2, fb2, g2, be2,
        wox, box, woy, boy, waw, baw):
    wspec = lambda shape: pl.BlockSpec(shape, lambda b: tuple(0 for _ in shape))
    return pl.pallas_call(
        _t2_body,
        grid=(BS,),
        in_specs=[
            pl.BlockSpec((1, K, C), lambda b: (b, 0, 0)),
            pl.BlockSpec((1, K, C), lambda b: (b, 0, 0)),
            pl.BlockSpec((1, 1, K), lambda b: (b, 0, 0)),
            pl.BlockSpec((1, K, C), lambda b: (b, 0, 0)),
            wspec((C, C)), wspec((1, C)),
            wspec((1, C)), wspec((1, C)),
            wspec((C, DFF)), wspec((1, DFF)), wspec((DFF, C)), wspec((1, C)),
            wspec((1, C)), wspec((1, C)),
            wspec((C, LPG)), wspec((1, LPG)), wspec((C, LPG)), wspec((1, LPG)),
            wspec((C, LPG)), wspec((1, LPG)),
        ],
        out_specs=[
            pl.BlockSpec((1, K, C), lambda b: (b, 0, 0)),
            pl.BlockSpec((1, K, RPT), lambda b: (b, 0, 0)),
            pl.BlockSpec((1, K, RPT), lambda b: (b, 0, 0)),
        ],
        out_shape=[
            jax.ShapeDtypeStruct((BS, K, C), jnp.float32),
            jax.ShapeDtypeStruct((BS, K, RPT), jnp.int32),
            jax.ShapeDtypeStruct((BS, K, RPT), jnp.float32),
        ],
    )(query, qpos, ind3, acc, wo, bo, g1, be1, f1, fb1, f2, fb2, g2, be2,
      wox, box, woy, boy, waw, baw)


def _sc_sample(vtab, idx, wgt):
    mesh = plsc.VectorSubcoreMesh(core_axis_name="c", subcore_axis_name="s")

    NBUF = 4

    @functools.partial(
        pl.kernel, mesh=mesh,
        out_type=jax.ShapeDtypeStruct((BS * K, NH, DH), jnp.float32),
        compiler_params=pltpu.CompilerParams(use_tc_tiling_on_sc=False),
        scratch_types=[
            pltpu.VMEM((NBUF, NCORNER, 128), jnp.int32),
            pltpu.VMEM((NBUF, RPT), jnp.float32),
            pltpu.VMEM((NBUF, RPT, DH // 2), jnp.int32),
            pltpu.VMEM((NH, DH), jnp.float32),
        ] + [pltpu.SemaphoreType.DMA] * NBUF,
    )
    def k(vtab_hbm, idx_hbm, wgt_hbm, out_hbm, idx_v, wgt_v, rows_v,
          stage_v, *sems):
        wid = lax.axis_index("s") * 2 + lax.axis_index("c")
        o0 = wid * PER_W

        def fire(o, s):
            pltpu.sync_copy(idx_hbm.at[o], idx_v.at[s])
            pltpu.sync_copy(wgt_hbm.at[o], wgt_v.at[s])
            for j in range(NCORNER):
                pltpu.async_copy(vtab_hbm.at[idx_v.at[s, j]],
                                 rows_v.at[s, pl.ds(j * 128, 128)], sems[s])

        def drain(s):
            for j in range(NCORNER):
                pltpu.make_async_copy(vtab_hbm.at[idx_v.at[s, j]],
                                      rows_v.at[s, pl.ds(j * 128, 128)],
                                      sems[s]).wait()

        def compute(o, s):
            for h in range(NH):
                acc0 = jnp.zeros((16,), jnp.float32)
                acc1 = jnp.zeros((16,), jnp.float32)
                for c in range(NCORNER):
                    wv = wgt_v[s, c * 128 + h * 16:c * 128 + h * 16 + 16]
                    for t in range(L * NP):
                        r = c * 128 + h * 16 + t
                        w = wv[t]
                        ri = rows_v[s, r, :]
                        ra = lax.bitcast_convert_type(
                            lax.shift_left(ri, 16), jnp.float32)
                        rb = lax.bitcast_convert_type(
                            lax.bitwise_and(ri, jnp.int32(-65536)), jnp.float32)
                        acc0 = acc0 + w * ra
                        acc1 = acc1 + w * rb
                stage_v[h, 0:16] = acc0
                stage_v[h, 16:32] = acc1
            pltpu.sync_copy(stage_v, out_hbm.at[o])

        for s in range(NBUF):
            fire(o0 + s, s)

        def body(j, _):
            o = o0 + NBUF * j
            for s in range(NBUF):
                drain(s)
                compute(o + s, s)
                pl.when(j < PER_W // NBUF - 1)(
                    functools.partial(fire, o + s + NBUF, s))
            return 0

        lax.fori_loop(0, PER_W // NBUF, body, 0)

    return k(vtab, idx, wgt)


_PB = 2048


def _scatter_body(ind_ref, q_ref, out_ref):
    j = pl.program_id(1)
    ind = ind_ref[0, 0, :].reshape(K, 1)
    cols = lax.broadcasted_iota(jnp.int32, (K, _PB), 1) + j * _PB
    oh = (cols == ind).astype(jnp.float32)
    out_ref[0] = lax.dot_general(q_ref[0], oh, (((0,), (0,)), ((), ())),
                                 preferred_element_type=jnp.float32)


def _scatter_fill(query, ind3):
    return pl.pallas_call(
        _scatter_body,
        grid=(BS, HW // _PB),
        in_specs=[
            pl.BlockSpec((1, 1, K), lambda b, j: (b, 0, 0)),
            pl.BlockSpec((1, K, C), lambda b, j: (b, 0, 0)),
        ],
        out_specs=pl.BlockSpec((1, C, _PB), lambda b, j: (b, 0, j)),
        out_shape=jax.ShapeDtypeStruct((BS, C, HW), jnp.float32),
    )(ind3, query)


def kernel(x, preds, mlp_w1, mlp_b1, mlp_w2, mlp_b2, time_emb, Wv, bv, Woff,
           boff, Waw, baw, Wo, bo, ln1_g, ln1_b, ffn_w1, ffn_b1, ffn_w2,
           ffn_b2, ln2_g, ln2_b):
    x4 = x.reshape(BS, L, C, HW)
    woff_r = Woff.reshape(NLAYERS, C, NH, L, NP, 2)
    wox = woff_r[..., 0].reshape(NLAYERS, C, LPG)
    woy = woff_r[..., 1].reshape(NLAYERS, C, LPG)
    boff_r = boff.reshape(NLAYERS, NH, L, NP, 2)
    box = boff_r[..., 0].reshape(NLAYERS, 1, LPG)
    boy = boff_r[..., 1].reshape(NLAYERS, 1, LPG)
    vbias = (jnp.einsum('lc,ncd->nld', time_emb, Wv)
             + bv[:, None, :]).reshape(NLAYERS, L, 1, C)
    x4b = x4.astype(jnp.bfloat16)
    Wvb = Wv.astype(jnp.bfloat16)
    w1p = jnp.concatenate([mlp_w1, jnp.zeros((10, C), jnp.float32)], 0)
    b1 = mlp_b1.reshape(1, C)
    b2 = mlp_b2.reshape(1, C)
    baw_r = baw.reshape(NLAYERS, 1, LPG)
    bo_r = bo.reshape(NLAYERS, 1, C)
    ln1g_r = ln1_g.reshape(NLAYERS, 1, C); ln1b_r = ln1_b.reshape(NLAYERS, 1, C)
    ln2g_r = ln2_g.reshape(NLAYERS, 1, C); ln2b_r = ln2_b.reshape(NLAYERS, 1, C)
    fb1_r = ffn_b1.reshape(NLAYERS, 1, DFF); fb2_r = ffn_b2.reshape(NLAYERS, 1, C)

    hm = []; start = 0
    for cc in CLASS_COUNTS:
        hm.append(preds[:, start + 10:start + 10 + cc])
        start += 10 + cc
    heat = jnp.clip(jax.nn.sigmoid(jnp.concatenate(hm, 1)), 1e-4, 1.0 - 1e-4)
    hmask = jnp.max(heat, axis=1).reshape(BS, -1)
    _, mask_ind = lax.top_k(hmask, K)
    ind3 = mask_ind.reshape(BS, 1, K)
    q_all = preds.reshape(BS, 70, -1).transpose(0, 2, 1)
    qsel = jnp.take_along_axis(
        q_all, jnp.broadcast_to(mask_ind[:, :, None], (BS, K, 70)), axis=1)
    qsel80 = jnp.concatenate([qsel, jnp.zeros((BS, K, 10), jnp.float32)], 2)
    qpos = jnp.asarray(_qpos_table())[mask_ind]

    query, idx, wgt = _t1(qsel80, qpos, ind3, w1p, b1, mlp_w2, b2,
                          wox[0], box[0], woy[0], boy[0], Waw[0], baw_r[0])
    vts = _v_project(x4b, Wvb, vbias)
    for lyr in range(NLAYERS):
        vt = vts[lyr].reshape(BS * L * NH * HW, DH // 2)
        acc = _sc_sample(vt, idx.reshape(BS * K, NCORNER, 128),
                         wgt.reshape(BS * K, RPT))
        nxt = min(lyr + 1, NLAYERS - 1)
        query, idx, wgt = _t2(
            query, qpos, ind3, acc.reshape(BS, K, C), Wo[lyr], bo_r[lyr],
            ln1g_r[lyr], ln1b_r[lyr], ffn_w1[lyr], fb1_r[lyr], ffn_w2[lyr],
            fb2_r[lyr], ln2g_r[lyr], ln2b_r[lyr], wox[nxt], box[nxt],
            woy[nxt], boy[nxt], Waw[nxt], baw_r[nxt])
    fill = _scatter_fill(query, ind3)
    return fill.reshape(BS, C, H, W)
